# baseline (speedup 1.0000x reference)
import jax
import jax.numpy as jnp
from jax import lax
from jax.experimental import pallas as pl
from jax.experimental.pallas import tpu as pltpu
from jax.experimental.pallas import tpu_sc as plsc

DIM_ = 512
D2 = 1024
KC = 128
NB = 2048

NWORK = 32
BPW = NB // NWORK


def _sc_gather_body(table_hbm, idx_hbm, out_hbm, idx_v, rows_v, sem):
    wid = lax.axis_index("s") * 2 + lax.axis_index("c")
    base = wid * BPW
    pltpu.sync_copy(idx_hbm.at[pl.ds(base, BPW)], idx_v)
    pltpu.async_copy(table_hbm.at[idx_v], rows_v, sem).wait()
    pltpu.sync_copy(rows_v, out_hbm.at[pl.ds(base, BPW)])


def _sc_gather(table, idx):
    mesh = plsc.VectorSubcoreMesh(core_axis_name="c", subcore_axis_name="s")
    k = pl.kernel(
        _sc_gather_body,
        mesh=mesh,
        out_type=jax.ShapeDtypeStruct((NB, D2), jnp.float32),
        scratch_types=[
            pltpu.VMEM((BPW,), jnp.int32),
            pltpu.VMEM((BPW, D2), jnp.float32),
            pltpu.SemaphoreType.DMA,
        ],
    )
    return k(table, idx)


BM = 256


def _stage_body(gate_ref, emb_ref, wqkv_ref, bqkv_ref, ngam_ref, nbet_ref,
                vqe_ref, ws_ref, bs_ref, visp_ref, audp_ref, wc_ref, bc_ref,
                f_ref):
    scale = 1.0 - jax.nn.sigmoid(gate_ref[0, 0])
    gw = emb_ref[...] * scale
    gwb = gw.astype(jnp.bfloat16)
    qkv = jnp.dot(gwb, wqkv_ref[...], preferred_element_type=jnp.float32)
    qkv = qkv + bqkv_ref[...]
    q = qkv[:, :D2]
    k = qkv[:, D2:2 * D2]
    v = qkv[:, 2 * D2:]
    score = jnp.sum(q * k, axis=-1, keepdims=True)
    g = v * jax.nn.sigmoid(score)

    def _ln(x):
        m = jnp.mean(x, axis=-1, keepdims=True)
        var = jnp.mean((x - m) ** 2, axis=-1, keepdims=True)
        return (x - m) * lax.rsqrt(var + 1e-5)

    zf = jnp.concatenate([_ln(g[:, :DIM_]), _ln(g[:, DIM_:])], axis=-1)
    zf = zf * ngam_ref[...] + nbet_ref[...]

    vqe = vqe_ref[...]
    t = lax.dot_general(zf.astype(jnp.bfloat16), vqe.astype(jnp.bfloat16),
                        (((1,), (1,)), ((), ())),
                        preferred_element_type=jnp.float32)
    ones = jnp.ones((1, D2), jnp.float32)
    ysq = lax.dot_general(ones, vqe * vqe, (((1,), (1,)), ((), ())),
                          preferred_element_type=jnp.float32)
    d = ysq - 2.0 * t
    dmin = jnp.min(d, axis=-1, keepdims=True)
    iot = lax.broadcasted_iota(jnp.int32, (BM, KC), 1)
    am = jnp.min(jnp.where(d <= dmin, iot, KC), axis=-1, keepdims=True)
    oh = (iot == am).astype(jnp.float32)
    zq = jnp.dot(oh, vqe, preferred_element_type=jnp.float32)

    sflat = jnp.dot(zq.astype(jnp.bfloat16), ws_ref[...],
                    preferred_element_type=jnp.float32) + bs_ref[...]

    def _palette(pal):
        logit = lax.dot_general(sflat, pal, (((1,), (1,)), ((), ())),
                                preferred_element_type=jnp.float32)
        logit = logit - jnp.max(logit, axis=-1, keepdims=True)
        e = jnp.exp(logit)
        attn = e / jnp.sum(e, axis=-1, keepdims=True)
        return jnp.dot(attn, pal, preferred_element_type=jnp.float32)

    vo = _palette(visp_ref[...])
    ao = _palette(audp_ref[...])

    cc = jnp.dot(zf.astype(jnp.bfloat16), wc_ref[...],
                 preferred_element_type=jnp.float32) + bc_ref[...]

    fr = zf[:, :DIM_] + (vo[:, :DIM_] - ao[:, DIM_:]) - cc[:, DIM_:]
    fi = zf[:, DIM_:] + (vo[:, DIM_:] + ao[:, :DIM_]) + cc[:, :DIM_]
    f_ref[...] = jnp.concatenate([fr, fi], axis=-1)


def _stage_call(gate2d, emb, wqkv, bqkv, ngam, nbet, vqe, ws, bs, visp, audp,
                wc, bc, interpret=False):
    ni = NB // BM
    const = lambda shape: pl.BlockSpec(shape, lambda i: (0, 0))
    return pl.pallas_call(
        _stage_body,
        grid=(ni,),
        in_specs=[
            pl.BlockSpec((1, 1), lambda i: (0, 0), memory_space=pltpu.SMEM),
            pl.BlockSpec((BM, D2), lambda i: (i, 0)),
            const((D2, 3 * D2)),
            const((1, 3 * D2)),
            const((1, D2)),
            const((1, D2)),
            const((KC, D2)),
            const((D2, D2)),
            const((1, D2)),
            const((32, D2)),
            const((32, D2)),
            const((D2, D2)),
            const((1, D2)),
        ],
        out_specs=pl.BlockSpec((BM, D2), lambda i: (i, 0)),
        out_shape=jax.ShapeDtypeStruct((NB, D2), jnp.float32),
        compiler_params=pltpu.CompilerParams(
            dimension_semantics=("arbitrary",)),
        interpret=interpret,
    )(gate2d, emb, wqkv, bqkv, ngam, nbet, vqe, ws, bs, visp, audp, wc, bc)


BMD = 512
BND = 1024


def _dec_body(f_ref, dw_ref, db_ref, out_ref):
    out_ref[...] = jnp.dot(f_ref[...].astype(jnp.bfloat16), dw_ref[...],
                           preferred_element_type=jnp.float32) + db_ref[...]


def _dec_call(f, dw16, db2d, interpret=False):
    nj = 8192 // BND
    ni = NB // BMD
    return pl.pallas_call(
        _dec_body,
        grid=(nj, ni),
        in_specs=[
            pl.BlockSpec((BMD, D2), lambda j, i: (i, 0)),
            pl.BlockSpec((D2, BND), lambda j, i: (0, j)),
            pl.BlockSpec((1, BND), lambda j, i: (0, j)),
        ],
        out_specs=pl.BlockSpec((BMD, BND), lambda j, i: (i, j)),
        out_shape=jax.ShapeDtypeStruct((NB, 8192), jnp.float32),
        compiler_params=pltpu.CompilerParams(
            dimension_semantics=("arbitrary", "arbitrary")),
        interpret=interpret,
    )(f, dw16, db2d)


def _stack(wr, wi):
    return jnp.concatenate([jnp.concatenate([wr, wi], 1),
                            jnp.concatenate([-wi, wr], 1)], 0)


def _bias2(br, bi):
    return jnp.concatenate([br - bi, br + bi])[None, :]


def kernel(tokens, enc_table, input_gate,
           qWr, qbr, qWi, qbi,
           kWr, kbr, kWi, kbi,
           vWr, vbr, vWi, vbi,
           sWr, sbr, sWi, sbi,
           cWr, cbr, cWi, cbi,
           mgW, mgb, maW, mab,
           mn_gr, mn_br, mn_gi, mn_bi,
           n_gr, n_br, n_gi, n_bi,
           vq_E, vis_P, aud_P, dW, db):
    emb = _sc_gather(enc_table, tokens.astype(jnp.int32))

    wqkv = jnp.concatenate(
        [_stack(qWr, qWi), _stack(kWr, kWi), _stack(vWr, vWi)],
        axis=1).astype(jnp.bfloat16)
    bqkv = jnp.concatenate(
        [_bias2(qbr, qbi), _bias2(kbr, kbi), _bias2(vbr, vbi)], axis=1)
    ws = _stack(sWr, sWi).astype(jnp.bfloat16)
    bs = _bias2(sbr, sbi)
    wc = _stack(cWr, cWi).astype(jnp.bfloat16)
    bc = _bias2(cbr, cbi)
    ngam = jnp.concatenate([n_gr, n_gi])[None, :]
    nbet = jnp.concatenate([n_br, n_bi])[None, :]
    gate2d = jnp.reshape(input_gate, (1, 1)).astype(jnp.float32)

    f = _stage_call(gate2d, emb, wqkv, bqkv, ngam, nbet, vq_E, ws, bs,
                    vis_P, aud_P, wc, bc)
    logits = _dec_call(f, dW.astype(jnp.bfloat16), db[None, :])
    return logits


# speedup vs baseline: 1.1189x; 1.1189x over previous
import jax
import jax.numpy as jnp
from jax import lax
from jax.experimental import pallas as pl
from jax.experimental.pallas import tpu as pltpu
from jax.experimental.pallas import tpu_sc as plsc

DIM_ = 512
D2 = 1024
KC = 128
NB = 2048

NWORK = 32
BPW = NB // NWORK


def _sc_gather_body(table_hbm, idx_hbm, out_hbm, idx_v, rows_v, sem):
    wid = lax.axis_index("s") * 2 + lax.axis_index("c")
    base = wid * BPW
    pltpu.sync_copy(idx_hbm.at[pl.ds(base, BPW)], idx_v)
    pltpu.async_copy(table_hbm.at[idx_v], rows_v, sem).wait()
    pltpu.sync_copy(rows_v, out_hbm.at[pl.ds(base, BPW)])


def _sc_gather(table, idx):
    mesh = plsc.VectorSubcoreMesh(core_axis_name="c", subcore_axis_name="s")
    k = pl.kernel(
        _sc_gather_body,
        mesh=mesh,
        out_type=jax.ShapeDtypeStruct((NB, D2), jnp.float32),
        scratch_types=[
            pltpu.VMEM((BPW,), jnp.int32),
            pltpu.VMEM((BPW, D2), jnp.float32),
            pltpu.SemaphoreType.DMA,
        ],
    )
    return k(table, idx)


BM = 256


def _stage_body(gate_ref, emb_ref, wqkv_ref, bqkv_ref, ngam_ref, nbet_ref,
                vqe_ref, ws_ref, bs_ref, visp_ref, audp_ref, wc_ref, bc_ref,
                f_ref):
    scale = 1.0 - jax.nn.sigmoid(gate_ref[0, 0])
    gw = emb_ref[...] * scale
    gwb = gw.astype(jnp.bfloat16)
    qkv = jnp.dot(gwb, wqkv_ref[...], preferred_element_type=jnp.float32)
    qkv = qkv + bqkv_ref[...]
    q = qkv[:, :D2]
    k = qkv[:, D2:2 * D2]
    v = qkv[:, 2 * D2:]
    score = jnp.sum(q * k, axis=-1, keepdims=True)
    g = v * jax.nn.sigmoid(score)

    def _ln(x):
        m = jnp.mean(x, axis=-1, keepdims=True)
        var = jnp.mean((x - m) ** 2, axis=-1, keepdims=True)
        return (x - m) * lax.rsqrt(var + 1e-5)

    zf = jnp.concatenate([_ln(g[:, :DIM_]), _ln(g[:, DIM_:])], axis=-1)
    zf = zf * ngam_ref[...] + nbet_ref[...]

    vqe = vqe_ref[...]
    t = lax.dot_general(zf.astype(jnp.bfloat16), vqe.astype(jnp.bfloat16),
                        (((1,), (1,)), ((), ())),
                        preferred_element_type=jnp.float32)
    ones = jnp.ones((1, D2), jnp.float32)
    ysq = lax.dot_general(ones, vqe * vqe, (((1,), (1,)), ((), ())),
                          preferred_element_type=jnp.float32)
    d = ysq - 2.0 * t
    dmin = jnp.min(d, axis=-1, keepdims=True)
    iot = lax.broadcasted_iota(jnp.int32, (BM, KC), 1)
    am = jnp.min(jnp.where(d <= dmin, iot, KC), axis=-1, keepdims=True)
    oh = (iot == am).astype(jnp.bfloat16)
    zq = jnp.dot(oh, vqe.astype(jnp.bfloat16),
                 preferred_element_type=jnp.float32)

    sflat = jnp.dot(zq.astype(jnp.bfloat16), ws_ref[...],
                    preferred_element_type=jnp.float32) + bs_ref[...]

    def _palette(pal):
        logit = lax.dot_general(sflat.astype(jnp.bfloat16),
                                pal.astype(jnp.bfloat16),
                                (((1,), (1,)), ((), ())),
                                preferred_element_type=jnp.float32)
        logit = logit - jnp.max(logit, axis=-1, keepdims=True)
        e = jnp.exp(logit)
        attn = e / jnp.sum(e, axis=-1, keepdims=True)
        return jnp.dot(attn.astype(jnp.bfloat16), pal.astype(jnp.bfloat16),
                       preferred_element_type=jnp.float32)

    vo = _palette(visp_ref[...])
    ao = _palette(audp_ref[...])

    cc = jnp.dot(zf.astype(jnp.bfloat16), wc_ref[...],
                 preferred_element_type=jnp.float32) + bc_ref[...]

    fr = zf[:, :DIM_] + (vo[:, :DIM_] - ao[:, DIM_:]) - cc[:, DIM_:]
    fi = zf[:, DIM_:] + (vo[:, DIM_:] + ao[:, :DIM_]) + cc[:, :DIM_]
    f_ref[...] = jnp.concatenate([fr, fi], axis=-1).astype(jnp.bfloat16)


def _stage_call(gate2d, emb, wqkv, bqkv, ngam, nbet, vqe, ws, bs, visp, audp,
                wc, bc, interpret=False):
    ni = NB // BM
    const = lambda shape: pl.BlockSpec(shape, lambda i: (0, 0))
    return pl.pallas_call(
        _stage_body,
        grid=(ni,),
        in_specs=[
            pl.BlockSpec((1, 1), lambda i: (0, 0), memory_space=pltpu.SMEM),
            pl.BlockSpec((BM, D2), lambda i: (i, 0)),
            const((D2, 3 * D2)),
            const((1, 3 * D2)),
            const((1, D2)),
            const((1, D2)),
            const((KC, D2)),
            const((D2, D2)),
            const((1, D2)),
            const((32, D2)),
            const((32, D2)),
            const((D2, D2)),
            const((1, D2)),
        ],
        out_specs=pl.BlockSpec((BM, D2), lambda i: (i, 0)),
        out_shape=jax.ShapeDtypeStruct((NB, D2), jnp.bfloat16),
        compiler_params=pltpu.CompilerParams(
            dimension_semantics=("arbitrary",)),
        interpret=interpret,
    )(gate2d, emb, wqkv, bqkv, ngam, nbet, vqe, ws, bs, visp, audp, wc, bc)


BND = 512


def _dec_body(f_ref, dw_ref, db_ref, out_ref):
    out_ref[...] = jnp.dot(f_ref[...], dw_ref[...],
                           preferred_element_type=jnp.float32) + db_ref[...]


def _dec_call(f, dw16, db2d, interpret=False):
    nj = 8192 // BND
    return pl.pallas_call(
        _dec_body,
        grid=(nj,),
        in_specs=[
            pl.BlockSpec((NB, D2), lambda j: (0, 0)),
            pl.BlockSpec((D2, BND), lambda j: (0, j)),
            pl.BlockSpec((1, BND), lambda j: (0, j)),
        ],
        out_specs=pl.BlockSpec((NB, BND), lambda j: (0, j)),
        out_shape=jax.ShapeDtypeStruct((NB, 8192), jnp.float32),
        compiler_params=pltpu.CompilerParams(
            dimension_semantics=("arbitrary",)),
        interpret=interpret,
    )(f, dw16, db2d)


def _stack(wr, wi):
    return jnp.concatenate([jnp.concatenate([wr, wi], 1),
                            jnp.concatenate([-wi, wr], 1)], 0)


def _bias2(br, bi):
    return jnp.concatenate([br - bi, br + bi])[None, :]


def kernel(tokens, enc_table, input_gate,
           qWr, qbr, qWi, qbi,
           kWr, kbr, kWi, kbi,
           vWr, vbr, vWi, vbi,
           sWr, sbr, sWi, sbi,
           cWr, cbr, cWi, cbi,
           mgW, mgb, maW, mab,
           mn_gr, mn_br, mn_gi, mn_bi,
           n_gr, n_br, n_gi, n_bi,
           vq_E, vis_P, aud_P, dW, db):
    emb = _sc_gather(enc_table, tokens.astype(jnp.int32))

    wqkv = jnp.concatenate(
        [_stack(qWr, qWi), _stack(kWr, kWi), _stack(vWr, vWi)],
        axis=1).astype(jnp.bfloat16)
    bqkv = jnp.concatenate(
        [_bias2(qbr, qbi), _bias2(kbr, kbi), _bias2(vbr, vbi)], axis=1)
    ws = _stack(sWr, sWi).astype(jnp.bfloat16)
    bs = _bias2(sbr, sbi)
    wc = _stack(cWr, cWi).astype(jnp.bfloat16)
    bc = _bias2(cbr, cbi)
    ngam = jnp.concatenate([n_gr, n_gi])[None, :]
    nbet = jnp.concatenate([n_br, n_bi])[None, :]
    gate2d = jnp.reshape(input_gate, (1, 1)).astype(jnp.float32)

    f = _stage_call(gate2d, emb, wqkv, bqkv, ngam, nbet, vq_E, ws, bs,
                    vis_P, aud_P, wc, bc)
    logits = _dec_call(f, dW.astype(jnp.bfloat16), db[None, :])
    return logits


# speedup vs baseline: 1.6419x; 1.4675x over previous
import jax
import jax.numpy as jnp
from jax import lax
from jax.experimental import pallas as pl
from jax.experimental.pallas import tpu as pltpu
from jax.experimental.pallas import tpu_sc as plsc

DIM_ = 512
D2 = 1024
KC = 128
NB = 2048

NWORK = 32
BPW = NB // NWORK


def _sc_gather_body(table_hbm, idx_hbm, out_hbm, idx_v, rows_v, sem):
    wid = lax.axis_index("s") * 2 + lax.axis_index("c")
    base = wid * BPW
    pltpu.sync_copy(idx_hbm.at[pl.ds(base, BPW)], idx_v)
    pltpu.async_copy(table_hbm.at[idx_v], rows_v, sem).wait()
    pltpu.sync_copy(rows_v, out_hbm.at[pl.ds(base, BPW)])


def _sc_gather(table, idx):
    mesh = plsc.VectorSubcoreMesh(core_axis_name="c", subcore_axis_name="s")
    k = pl.kernel(
        _sc_gather_body,
        mesh=mesh,
        out_type=jax.ShapeDtypeStruct((NB, D2), jnp.float32),
        scratch_types=[
            pltpu.VMEM((BPW,), jnp.int32),
            pltpu.VMEM((BPW, D2), jnp.float32),
            pltpu.SemaphoreType.DMA,
        ],
    )
    return k(table, idx)


BM = 256


def _dot(a, b):
    return jnp.dot(a, b, preferred_element_type=jnp.float32)


def _dot_t(a, b):
    return lax.dot_general(a, b, (((1,), (1,)), ((), ())),
                           preferred_element_type=jnp.float32)


def _stage_body(gate_ref, emb_ref,
                qwr, qwi, kwr, kwi, vwr, vwi,
                bq_ref, bk_ref, bv_ref,
                ngam_ref, nbet_ref, vqe_ref,
                swr, swi, bs_ref, visp_ref, audp_ref,
                cwr, cwi, bc_ref,
                f_ref):
    scale = 1.0 - jax.nn.sigmoid(gate_ref[0, 0])
    gw = emb_ref[...] * scale
    xr = gw[:, :DIM_].astype(jnp.bfloat16)
    xi = gw[:, DIM_:].astype(jnp.bfloat16)

    def _clin(ar, ai, wr, wi, bias):
        out_r = _dot(ar, wr[...]) - _dot(ai, wi[...]) + bias[:, :DIM_]
        out_i = _dot(ar, wi[...]) + _dot(ai, wr[...]) + bias[:, DIM_:]
        return out_r, out_i

    q_r, q_i = _clin(xr, xi, qwr, qwi, bq_ref[...])
    k_r, k_i = _clin(xr, xi, kwr, kwi, bk_ref[...])
    v_r, v_i = _clin(xr, xi, vwr, vwi, bv_ref[...])
    score = jnp.sum(q_r * k_r + q_i * k_i, axis=-1, keepdims=True)
    gate = jax.nn.sigmoid(score)

    def _ln(x):
        m = jnp.mean(x, axis=-1, keepdims=True)
        var = jnp.mean((x - m) ** 2, axis=-1, keepdims=True)
        return (x - m) * lax.rsqrt(var + 1e-5)

    zr = _ln(v_r * gate)
    zi = _ln(v_i * gate)
    zf = jnp.concatenate([zr, zi], axis=-1)
    zf = zf * ngam_ref[...] + nbet_ref[...]

    vqe = vqe_ref[...]
    t = _dot_t(zf.astype(jnp.bfloat16), vqe.astype(jnp.bfloat16))
    ones = jnp.ones((1, D2), jnp.float32)
    ysq = _dot_t(ones, vqe * vqe)
    d = ysq - 2.0 * t
    dmin = jnp.min(d, axis=-1, keepdims=True)
    iot = lax.broadcasted_iota(jnp.int32, (BM, KC), 1)
    am = jnp.min(jnp.where(d <= dmin, iot, KC), axis=-1, keepdims=True)
    oh = (iot == am).astype(jnp.bfloat16)
    zq = _dot(oh, vqe.astype(jnp.bfloat16))

    zqr = zq[:, :DIM_].astype(jnp.bfloat16)
    zqi = zq[:, DIM_:].astype(jnp.bfloat16)
    s_r, s_i = _clin(zqr, zqi, swr, swi, bs_ref[...])
    sflat = jnp.concatenate([s_r, s_i], axis=-1).astype(jnp.bfloat16)

    def _palette(pal_ref):
        pal = pal_ref[...].astype(jnp.bfloat16)
        logit = _dot_t(sflat, pal)
        logit = logit - jnp.max(logit, axis=-1, keepdims=True)
        e = jnp.exp(logit)
        attn = e / jnp.sum(e, axis=-1, keepdims=True)
        return _dot(attn.astype(jnp.bfloat16), pal)

    vo = _palette(visp_ref)
    ao = _palette(audp_ref)

    c_r, c_i = _clin(zf[:, :DIM_].astype(jnp.bfloat16),
                     zf[:, DIM_:].astype(jnp.bfloat16),
                     cwr, cwi, bc_ref[...])

    fr = zf[:, :DIM_] + (vo[:, :DIM_] - ao[:, DIM_:]) - c_i
    fi = zf[:, DIM_:] + (vo[:, DIM_:] + ao[:, :DIM_]) + c_r
    f_ref[...] = jnp.concatenate([fr, fi], axis=-1).astype(jnp.bfloat16)


def _stage_call(gate2d, emb, qwr, qwi, kwr, kwi, vwr, vwi, bq, bk, bv,
                ngam, nbet, vqe, swr, swi, bs, visp, audp, cwr, cwi, bc,
                interpret=False):
    ni = NB // BM
    const = lambda shape: pl.BlockSpec(shape, lambda i: (0, 0))
    w = const((DIM_, DIM_))
    b = const((1, D2))
    return pl.pallas_call(
        _stage_body,
        grid=(ni,),
        in_specs=[
            pl.BlockSpec((1, 1), lambda i: (0, 0), memory_space=pltpu.SMEM),
            pl.BlockSpec((BM, D2), lambda i: (i, 0)),
            w, w, w, w, w, w,
            b, b, b,
            b, b,
            const((KC, D2)),
            w, w, b,
            const((32, D2)),
            const((32, D2)),
            w, w, b,
        ],
        out_specs=pl.BlockSpec((BM, D2), lambda i: (i, 0)),
        out_shape=jax.ShapeDtypeStruct((NB, D2), jnp.bfloat16),
        compiler_params=pltpu.CompilerParams(
            dimension_semantics=("arbitrary",)),
        interpret=interpret,
    )(gate2d, emb, qwr, qwi, kwr, kwi, vwr, vwi, bq, bk, bv,
      ngam, nbet, vqe, swr, swi, bs, visp, audp, cwr, cwi, bc)


BND = 512


def _dec_body(f_ref, dw_ref, db_ref, out_ref):
    out_ref[...] = jnp.dot(f_ref[...], dw_ref[...].astype(jnp.bfloat16),
                           preferred_element_type=jnp.float32) + db_ref[...]


def _dec_call(f, dw, db2d, interpret=False):
    nj = 8192 // BND
    return pl.pallas_call(
        _dec_body,
        grid=(nj,),
        in_specs=[
            pl.BlockSpec((NB, D2), lambda j: (0, 0)),
            pl.BlockSpec((D2, BND), lambda j: (0, j)),
            pl.BlockSpec((1, BND), lambda j: (0, j)),
        ],
        out_specs=pl.BlockSpec((NB, BND), lambda j: (0, j)),
        out_shape=jax.ShapeDtypeStruct((NB, 8192), jnp.float32),
        compiler_params=pltpu.CompilerParams(
            dimension_semantics=("arbitrary",)),
        interpret=interpret,
    )(f, dw, db2d)


def _bias2(br, bi):
    return jnp.concatenate([br - bi, br + bi])[None, :]


def kernel(tokens, enc_table, input_gate,
           qWr, qbr, qWi, qbi,
           kWr, kbr, kWi, kbi,
           vWr, vbr, vWi, vbi,
           sWr, sbr, sWi, sbi,
           cWr, cbr, cWi, cbi,
           mgW, mgb, maW, mab,
           mn_gr, mn_br, mn_gi, mn_bi,
           n_gr, n_br, n_gi, n_bi,
           vq_E, vis_P, aud_P, dW, db):
    emb = _sc_gather(enc_table, tokens.astype(jnp.int32))

    b16 = lambda w: w.astype(jnp.bfloat16)
    ngam = jnp.concatenate([n_gr, n_gi])[None, :]
    nbet = jnp.concatenate([n_br, n_bi])[None, :]
    gate2d = jnp.reshape(input_gate, (1, 1)).astype(jnp.float32)

    f = _stage_call(gate2d, emb,
                    b16(qWr), b16(qWi), b16(kWr), b16(kWi), b16(vWr),
                    b16(vWi), _bias2(qbr, qbi), _bias2(kbr, kbi),
                    _bias2(vbr, vbi), ngam, nbet, vq_E,
                    b16(sWr), b16(sWi), _bias2(sbr, sbi), vis_P, aud_P,
                    b16(cWr), b16(cWi), _bias2(cbr, cbi))
    logits = _dec_call(f, dW, db[None, :])
    return logits
